# Initial kernel scaffold; baseline (speedup 1.0000x reference)
import jax
import jax.numpy as jnp
from jax.experimental import pallas as pl


def kernel(inputs, edge_index, W1, b1, W2, b2, W3, b3, Wfc, bfc):
    raise NotImplementedError("write your pallas kernel here")


# speedup vs baseline: 18.4749x; 18.4749x over previous
import functools

import jax
import jax.numpy as jnp
from jax import lax
from jax.experimental import pallas as pl
from jax.experimental.pallas import tpu as pltpu
from jax.experimental.pallas import tpu_sc as plsc

NC, NS = 2, 16
NW = NC * NS
NPAD = 10240
ZR = NPAD // NS
HW = 16
F = 32


def _sc_mesh():
    return plsc.VectorSubcoreMesh(
        core_axis_name="c", subcore_axis_name="s", num_cores=NC, num_subcores=NS
    )


_SC_PARAMS = pltpu.CompilerParams(use_tc_tiling_on_sc=False)


def _hist(src, dst):
    E = src.shape[0]
    EPW = E // NW
    K = 2000 if EPW % 2000 == 0 else EPW

    @functools.partial(
        pl.kernel,
        out_type=[
            jax.ShapeDtypeStruct((NC * NPAD, HW), jnp.float32),
            jax.ShapeDtypeStruct((NC * NPAD, HW), jnp.float32),
        ],
        mesh=_sc_mesh(),
        compiler_params=_SC_PARAMS,
        scratch_types=[
            pltpu.VMEM((K,), jnp.int32),
            pltpu.VMEM((K, HW), jnp.float32),
            pltpu.VMEM((ZR, HW), jnp.float32),
            pltpu.VMEM_SHARED((NPAD, HW), jnp.float32),
            pltpu.VMEM_SHARED((NPAD, HW), jnp.float32),
        ],
    )
    def k(src_h, dst_h, out_o, out_i, idx, ones, zbuf, acc_o, acc_i):
        cid = lax.axis_index("c")
        sid = lax.axis_index("s")
        wid = sid * NC + cid

        def fill(i, _):
            ones[i, :] = jnp.ones((HW,), jnp.float32)
            zbuf[i % ZR, :] = jnp.zeros((HW,), jnp.float32)
            return 0

        lax.fori_loop(0, K, fill, 0)
        pltpu.sync_copy(zbuf, acc_o.at[pl.ds(sid * ZR, ZR)])
        pltpu.sync_copy(zbuf, acc_i.at[pl.ds(sid * ZR, ZR)])
        plsc.subcore_barrier()

        base0 = wid * EPW

        def chunk(i, _):
            b = pl.multiple_of(base0 + i * K, 8)
            pltpu.sync_copy(src_h.at[pl.ds(b, K)], idx)
            pltpu.sync_copy(ones, acc_o.at[idx], add=True)
            pltpu.sync_copy(dst_h.at[pl.ds(b, K)], idx)
            pltpu.sync_copy(ones, acc_i.at[idx], add=True)
            return 0

        lax.fori_loop(0, EPW // K, chunk, 0)
        plsc.subcore_barrier()
        pltpu.sync_copy(acc_o.at[pl.ds(sid * ZR, ZR)],
                        out_o.at[pl.ds(cid * NPAD + sid * ZR, ZR)])
        pltpu.sync_copy(acc_i.at[pl.ds(sid * ZR, ZR)],
                        out_i.at[pl.ds(cid * NPAD + sid * ZR, ZR)])

    return k(src, dst)


def _edge_agg(table, src, dst):
    E = src.shape[0]
    EPW = E // NW
    K = 2000 if EPW % 2000 == 0 else EPW

    @functools.partial(
        pl.kernel,
        out_type=jax.ShapeDtypeStruct((NC * NPAD, F), jnp.float32),
        mesh=_sc_mesh(),
        compiler_params=_SC_PARAMS,
        scratch_types=[
            pltpu.VMEM((K,), jnp.int32),
            pltpu.VMEM((K,), jnp.int32),
            pltpu.VMEM((K, F), jnp.float32),
            pltpu.VMEM_SHARED((NPAD, F), jnp.float32),
            pltpu.SemaphoreType.DMA,
        ],
    )
    def k(table_h, src_h, dst_h, out_h, idx_s, idx_d, rows, acc, sem):
        cid = lax.axis_index("c")
        sid = lax.axis_index("s")
        wid = sid * NC + cid

        def zfill(i, _):
            rows[i, pl.ds(0, 16)] = jnp.zeros((16,), jnp.float32)
            rows[i, pl.ds(16, 16)] = jnp.zeros((16,), jnp.float32)
            return 0

        lax.fori_loop(0, ZR, zfill, 0)
        pltpu.sync_copy(rows.at[pl.ds(0, ZR)], acc.at[pl.ds(sid * ZR, ZR)])
        plsc.subcore_barrier()

        base0 = wid * EPW

        def chunk(i, _):
            b = pl.multiple_of(base0 + i * K, 8)
            pltpu.sync_copy(src_h.at[pl.ds(b, K)], idx_s)
            g = pltpu.async_copy(table_h.at[idx_s], rows, sem)
            pltpu.sync_copy(dst_h.at[pl.ds(b, K)], idx_d)
            g.wait()
            pltpu.sync_copy(rows, acc.at[idx_d], add=True)
            return 0

        lax.fori_loop(0, EPW // K, chunk, 0)
        plsc.subcore_barrier()
        pltpu.sync_copy(acc.at[pl.ds(sid * ZR, ZR)],
                        out_h.at[pl.ds(cid * NPAD + sid * ZR, ZR)])

    return k(table, src, dst)


def _prep(x, W1, dop, ddp):
    N = x.shape[0]

    def body(x_ref, w_ref, do_ref, di_ref, t_ref, no_ref, ni_ref):
        do = do_ref[0, :N, :] + do_ref[1, :N, :]
        di = di_ref[0, :N, :] + di_ref[1, :N, :]
        no = jnp.where(do > 0, lax.rsqrt(jnp.maximum(do, 1e-12)), 0.0)
        ni = jnp.where(di > 0, lax.rsqrt(jnp.maximum(di, 1e-12)), 0.0)
        no_ref[...] = no
        ni_ref[...] = ni
        y = jnp.dot(x_ref[...], w_ref[...], preferred_element_type=jnp.float32)
        t_ref[...] = y * no[:, :1]

    return pl.pallas_call(
        body,
        out_shape=[
            jax.ShapeDtypeStruct((N, F), jnp.float32),
            jax.ShapeDtypeStruct((N, HW), jnp.float32),
            jax.ShapeDtypeStruct((N, HW), jnp.float32),
        ],
    )(x, W1, dop, ddp)


def _mid(parts, ni, no, b, Wn):
    N = ni.shape[0]

    def body(p_ref, ni_ref, no_ref, b_ref, w_ref, t_ref):
        agg = p_ref[0, :N, :] + p_ref[1, :N, :]
        h = jnp.maximum(agg * ni_ref[...][:, :1] + b_ref[...], 0.0)
        y = jnp.dot(h, w_ref[...], preferred_element_type=jnp.float32)
        t_ref[...] = y * no_ref[...][:, :1]

    return pl.pallas_call(
        body,
        out_shape=jax.ShapeDtypeStruct((N, F), jnp.float32),
    )(parts, ni, no, b, Wn)


def _final(parts, ni, b, Wfc, bfc):
    N = ni.shape[0]

    def body(p_ref, ni_ref, b_ref, w_ref, bf_ref, o_ref):
        agg = p_ref[0, :N, :] + p_ref[1, :N, :]
        h = jnp.maximum(agg * ni_ref[...][:, :1] + b_ref[...], 0.0)
        pooled = jnp.max(h, axis=0, keepdims=True)
        logits = jnp.dot(pooled, w_ref[...],
                         preferred_element_type=jnp.float32) + bf_ref[...]
        m = jnp.max(logits, axis=-1, keepdims=True)
        e = jnp.exp(logits - m)
        o_ref[...] = e / jnp.sum(e, axis=-1, keepdims=True)

    return pl.pallas_call(
        body,
        out_shape=jax.ShapeDtypeStruct((1, bfc.shape[-1]), jnp.float32),
    )(parts, ni, b, Wfc, bfc)


def kernel(inputs, edge_index, W1, b1, W2, b2, W3, b3, Wfc, bfc):
    src = edge_index[0]
    dst = edge_index[1]

    d_out_p, d_in_p = _hist(src, dst)
    dop = d_out_p.reshape(NC, NPAD, HW)
    ddp = d_in_p.reshape(NC, NPAD, HW)

    table1, no, ni = _prep(inputs, W1, dop, ddp)
    p1 = _edge_agg(table1, src, dst).reshape(NC, NPAD, F)
    table2 = _mid(p1, ni, no, b1.reshape(1, -1), W2)
    p2 = _edge_agg(table2, src, dst).reshape(NC, NPAD, F)
    table3 = _mid(p2, ni, no, b2.reshape(1, -1), W3)
    p3 = _edge_agg(table3, src, dst).reshape(NC, NPAD, F)
    return _final(p3, ni, b3.reshape(1, -1), Wfc, bfc.reshape(1, -1))


# speedup vs baseline: 20.9376x; 1.1333x over previous
import functools

import jax
import jax.numpy as jnp
from jax import lax
from jax.experimental import pallas as pl
from jax.experimental.pallas import tpu as pltpu
from jax.experimental.pallas import tpu_sc as plsc

NC, NS = 2, 16
NW = NC * NS
NPAD = 10240
ZR = NPAD // NS
HW = 16
F = 32


def _sc_mesh():
    return plsc.VectorSubcoreMesh(
        core_axis_name="c", subcore_axis_name="s", num_cores=NC, num_subcores=NS
    )


_SC_PARAMS = pltpu.CompilerParams(use_tc_tiling_on_sc=False)


def _hist(src, dst):
    E = src.shape[0]
    EPW = E // NW
    K = 2000 if EPW % 2000 == 0 else EPW
    CH = EPW // K

    @functools.partial(
        pl.kernel,
        out_type=[
            jax.ShapeDtypeStruct((NC * NPAD, HW), jnp.float32),
            jax.ShapeDtypeStruct((NC * NPAD, HW), jnp.float32),
        ],
        mesh=_sc_mesh(),
        compiler_params=_SC_PARAMS,
        scratch_types=[
            pltpu.VMEM((K,), jnp.int32),
            pltpu.VMEM((K,), jnp.int32),
            pltpu.VMEM((K,), jnp.int32),
            pltpu.VMEM((K,), jnp.int32),
            pltpu.VMEM((K, HW), jnp.float32),
            pltpu.VMEM((ZR, HW), jnp.float32),
            pltpu.VMEM_SHARED((NPAD, HW), jnp.float32),
            pltpu.VMEM_SHARED((NPAD, HW), jnp.float32),
            pltpu.SemaphoreType.DMA,
            pltpu.SemaphoreType.DMA,
            pltpu.SemaphoreType.DMA,
        ],
    )
    def k(src_h, dst_h, out_o, out_i, idx_a0, idx_a1, idx_b0, idx_b1,
          ones, zbuf, acc_o, acc_i, sem_o, sem_i, sem_w):
        cid = lax.axis_index("c")
        sid = lax.axis_index("s")
        wid = sid * NC + cid
        idx_a = [idx_a0, idx_a1]
        idx_b = [idx_b0, idx_b1]

        def fill(i, _):
            ones[i, :] = jnp.ones((HW,), jnp.float32)
            zbuf[i % ZR, :] = jnp.zeros((HW,), jnp.float32)
            return 0

        lax.fori_loop(0, K, fill, 0)
        pltpu.sync_copy(zbuf, acc_o.at[pl.ds(sid * ZR, ZR)])
        pltpu.sync_copy(zbuf, acc_i.at[pl.ds(sid * ZR, ZR)])
        plsc.subcore_barrier()

        base0 = wid * EPW

        def boff(i):
            return pl.multiple_of(base0 + i * K, 8)

        so = [None] * CH
        si = [None] * CH
        pltpu.sync_copy(src_h.at[pl.ds(boff(0), K)], idx_a[0])
        for i in range(CH):
            j = i & 1
            so[i] = pltpu.async_copy(ones, acc_o.at[idx_a[j]], sem_o,
                                     add=True)
            if i >= 2:
                si[i - 2].wait()
            pltpu.sync_copy(dst_h.at[pl.ds(boff(i), K)], idx_b[j])
            si[i] = pltpu.async_copy(ones, acc_i.at[idx_b[j]], sem_i,
                                     add=True)
            if i + 1 < CH:
                if i >= 1:
                    so[i - 1].wait()
                pltpu.sync_copy(src_h.at[pl.ds(boff(i + 1), K)],
                                idx_a[1 - j])
        if CH >= 2:
            so[CH - 2].wait()
            si[CH - 2].wait()
        so[CH - 1].wait()
        si[CH - 1].wait()
        plsc.subcore_barrier()
        w0 = pltpu.async_copy(acc_o.at[pl.ds(sid * ZR, ZR)],
                              out_o.at[pl.ds(cid * NPAD + sid * ZR, ZR)],
                              sem_w)
        w1 = pltpu.async_copy(acc_i.at[pl.ds(sid * ZR, ZR)],
                              out_i.at[pl.ds(cid * NPAD + sid * ZR, ZR)],
                              sem_w)
        w0.wait()
        w1.wait()

    return k(src, dst)


def _edge_agg(table, src, dst):
    E = src.shape[0]
    EPW = E // NW
    K = 1000 if EPW % 1000 == 0 else EPW
    CH = EPW // K

    @functools.partial(
        pl.kernel,
        out_type=jax.ShapeDtypeStruct((NC * NPAD, F), jnp.float32),
        mesh=_sc_mesh(),
        compiler_params=_SC_PARAMS,
        scratch_types=[
            pltpu.VMEM((K,), jnp.int32),
            pltpu.VMEM((K,), jnp.int32),
            pltpu.VMEM((K,), jnp.int32),
            pltpu.VMEM((K,), jnp.int32),
            pltpu.VMEM((K, F), jnp.float32),
            pltpu.VMEM((K, F), jnp.float32),
            pltpu.VMEM_SHARED((NPAD, F), jnp.float32),
            pltpu.SemaphoreType.DMA,
            pltpu.SemaphoreType.DMA,
        ],
    )
    def k(table_h, src_h, dst_h, out_h, idx_s0, idx_s1, idx_d0, idx_d1,
          rows0, rows1, acc, gsem, ssem):
        cid = lax.axis_index("c")
        sid = lax.axis_index("s")
        wid = sid * NC + cid
        idx_s = [idx_s0, idx_s1]
        idx_d = [idx_d0, idx_d1]
        rows = [rows0, rows1]

        def zfill(i, _):
            rows0[i, pl.ds(0, 16)] = jnp.zeros((16,), jnp.float32)
            rows0[i, pl.ds(16, 16)] = jnp.zeros((16,), jnp.float32)
            return 0

        lax.fori_loop(0, ZR, zfill, 0)
        pltpu.sync_copy(rows0.at[pl.ds(0, ZR)], acc.at[pl.ds(sid * ZR, ZR)])
        plsc.subcore_barrier()

        base0 = wid * EPW

        def boff(i):
            return pl.multiple_of(base0 + i * K, 8)

        g = [None] * CH
        s = [None] * CH
        pltpu.sync_copy(src_h.at[pl.ds(boff(0), K)], idx_s[0])
        g[0] = pltpu.async_copy(table_h.at[idx_s[0]], rows[0], gsem)
        for i in range(CH):
            j = i & 1
            pltpu.sync_copy(dst_h.at[pl.ds(boff(i), K)], idx_d[j])
            if i + 1 < CH:
                pltpu.sync_copy(src_h.at[pl.ds(boff(i + 1), K)],
                                idx_s[1 - j])
            g[i].wait()
            s[i] = pltpu.async_copy(rows[j], acc.at[idx_d[j]], ssem,
                                    add=True)
            if i + 1 < CH:
                if i >= 1:
                    s[i - 1].wait()
                g[i + 1] = pltpu.async_copy(table_h.at[idx_s[1 - j]],
                                            rows[1 - j], gsem)
        if CH >= 2:
            s[CH - 2].wait()
        s[CH - 1].wait()
        plsc.subcore_barrier()
        pltpu.sync_copy(acc.at[pl.ds(sid * ZR, ZR)],
                        out_h.at[pl.ds(cid * NPAD + sid * ZR, ZR)])

    return k(table, src, dst)


def _prep(x, W1, dop, ddp):
    N = x.shape[0]

    def body(x_ref, w_ref, do_ref, di_ref, t_ref, no_ref, ni_ref):
        do = do_ref[0, :N, :] + do_ref[1, :N, :]
        di = di_ref[0, :N, :] + di_ref[1, :N, :]
        no = jnp.where(do > 0, lax.rsqrt(jnp.maximum(do, 1e-12)), 0.0)
        ni = jnp.where(di > 0, lax.rsqrt(jnp.maximum(di, 1e-12)), 0.0)
        no_ref[...] = no
        ni_ref[...] = ni
        y = jnp.dot(x_ref[...], w_ref[...], preferred_element_type=jnp.float32)
        t_ref[...] = y * no[:, :1]

    return pl.pallas_call(
        body,
        out_shape=[
            jax.ShapeDtypeStruct((N, F), jnp.float32),
            jax.ShapeDtypeStruct((N, HW), jnp.float32),
            jax.ShapeDtypeStruct((N, HW), jnp.float32),
        ],
    )(x, W1, dop, ddp)


def _mid(parts, ni, no, b, Wn):
    N = ni.shape[0]

    def body(p_ref, ni_ref, no_ref, b_ref, w_ref, t_ref):
        agg = p_ref[0, :N, :] + p_ref[1, :N, :]
        h = jnp.maximum(agg * ni_ref[...][:, :1] + b_ref[...], 0.0)
        y = jnp.dot(h, w_ref[...], preferred_element_type=jnp.float32)
        t_ref[...] = y * no_ref[...][:, :1]

    return pl.pallas_call(
        body,
        out_shape=jax.ShapeDtypeStruct((N, F), jnp.float32),
    )(parts, ni, no, b, Wn)


def _final(parts, ni, b, Wfc, bfc):
    N = ni.shape[0]

    def body(p_ref, ni_ref, b_ref, w_ref, bf_ref, o_ref):
        agg = p_ref[0, :N, :] + p_ref[1, :N, :]
        h = jnp.maximum(agg * ni_ref[...][:, :1] + b_ref[...], 0.0)
        pooled = jnp.max(h, axis=0, keepdims=True)
        logits = jnp.dot(pooled, w_ref[...],
                         preferred_element_type=jnp.float32) + bf_ref[...]
        m = jnp.max(logits, axis=-1, keepdims=True)
        e = jnp.exp(logits - m)
        o_ref[...] = e / jnp.sum(e, axis=-1, keepdims=True)

    return pl.pallas_call(
        body,
        out_shape=jax.ShapeDtypeStruct((1, bfc.shape[-1]), jnp.float32),
    )(parts, ni, b, Wfc, bfc)


def kernel(inputs, edge_index, W1, b1, W2, b2, W3, b3, Wfc, bfc):
    src = edge_index[0]
    dst = edge_index[1]

    d_out_p, d_in_p = _hist(src, dst)
    dop = d_out_p.reshape(NC, NPAD, HW)
    ddp = d_in_p.reshape(NC, NPAD, HW)

    table1, no, ni = _prep(inputs, W1, dop, ddp)
    p1 = _edge_agg(table1, src, dst).reshape(NC, NPAD, F)
    table2 = _mid(p1, ni, no, b1.reshape(1, -1), W2)
    p2 = _edge_agg(table2, src, dst).reshape(NC, NPAD, F)
    table3 = _mid(p2, ni, no, b2.reshape(1, -1), W3)
    p3 = _edge_agg(table3, src, dst).reshape(NC, NPAD, F)
    return _final(p3, ni, b3.reshape(1, -1), Wfc, bfc.reshape(1, -1))


# speedup vs baseline: 21.7910x; 1.0408x over previous
import functools

import jax
import jax.numpy as jnp
from jax import lax
from jax.experimental import pallas as pl
from jax.experimental.pallas import tpu as pltpu
from jax.experimental.pallas import tpu_sc as plsc

NC, NS = 2, 16
NW = NC * NS
NPAD = 10240
ZR = NPAD // NS
HW = 16
F = 32


def _sc_mesh():
    return plsc.VectorSubcoreMesh(
        core_axis_name="c", subcore_axis_name="s", num_cores=NC, num_subcores=NS
    )


_SC_PARAMS = pltpu.CompilerParams(use_tc_tiling_on_sc=False)


def _hist(ei):
    E = ei.shape[1]
    EPW = E // NW
    K = 2000 if EPW % 2000 == 0 else EPW
    CH = EPW // K

    @functools.partial(
        pl.kernel,
        out_type=[
            jax.ShapeDtypeStruct((NC, NPAD, HW), jnp.float32),
            jax.ShapeDtypeStruct((NC, NPAD, HW), jnp.float32),
        ],
        mesh=_sc_mesh(),
        compiler_params=_SC_PARAMS,
        scratch_types=[
            pltpu.VMEM((K,), jnp.int32),
            pltpu.VMEM((K,), jnp.int32),
            pltpu.VMEM((K,), jnp.int32),
            pltpu.VMEM((K,), jnp.int32),
            pltpu.VMEM((K, HW), jnp.float32),
            pltpu.VMEM((ZR, HW), jnp.float32),
            pltpu.VMEM_SHARED((NPAD, HW), jnp.float32),
            pltpu.VMEM_SHARED((NPAD, HW), jnp.float32),
            pltpu.SemaphoreType.DMA,
            pltpu.SemaphoreType.DMA,
            pltpu.SemaphoreType.DMA,
        ],
    )
    def k(ei_h, out_o, out_i, idx_a0, idx_a1, idx_b0, idx_b1,
          ones, zbuf, acc_o, acc_i, sem_o, sem_i, sem_w):
        cid = lax.axis_index("c")
        sid = lax.axis_index("s")
        wid = sid * NC + cid
        idx_a = [idx_a0, idx_a1]
        idx_b = [idx_b0, idx_b1]

        def fill(i, _):
            ones[i, :] = jnp.ones((HW,), jnp.float32)
            zbuf[i % ZR, :] = jnp.zeros((HW,), jnp.float32)
            return 0

        lax.fori_loop(0, K, fill, 0)
        pltpu.sync_copy(zbuf, acc_o.at[pl.ds(sid * ZR, ZR)])
        pltpu.sync_copy(zbuf, acc_i.at[pl.ds(sid * ZR, ZR)])
        plsc.subcore_barrier()

        base0 = wid * EPW

        def boff(i):
            return pl.multiple_of(base0 + i * K, 8)

        so = [None] * CH
        si = [None] * CH
        pltpu.sync_copy(ei_h.at[0, pl.ds(boff(0), K)], idx_a[0])
        for i in range(CH):
            j = i & 1
            so[i] = pltpu.async_copy(ones, acc_o.at[idx_a[j]], sem_o,
                                     add=True)
            if i >= 2:
                si[i - 2].wait()
            pltpu.sync_copy(ei_h.at[1, pl.ds(boff(i), K)], idx_b[j])
            si[i] = pltpu.async_copy(ones, acc_i.at[idx_b[j]], sem_i,
                                     add=True)
            if i + 1 < CH:
                if i >= 1:
                    so[i - 1].wait()
                pltpu.sync_copy(ei_h.at[0, pl.ds(boff(i + 1), K)],
                                idx_a[1 - j])
        if CH >= 2:
            so[CH - 2].wait()
            si[CH - 2].wait()
        so[CH - 1].wait()
        si[CH - 1].wait()
        plsc.subcore_barrier()
        w0 = pltpu.async_copy(acc_o.at[pl.ds(sid * ZR, ZR)],
                              out_o.at[cid, pl.ds(sid * ZR, ZR)],
                              sem_w)
        w1 = pltpu.async_copy(acc_i.at[pl.ds(sid * ZR, ZR)],
                              out_i.at[cid, pl.ds(sid * ZR, ZR)],
                              sem_w)
        w0.wait()
        w1.wait()

    return k(ei)


def _edge_agg(table, ei):
    E = ei.shape[1]
    EPW = E // NW
    K = 1000 if EPW % 1000 == 0 else EPW
    CH = EPW // K

    @functools.partial(
        pl.kernel,
        out_type=jax.ShapeDtypeStruct((NC, NPAD, F), jnp.float32),
        mesh=_sc_mesh(),
        compiler_params=_SC_PARAMS,
        scratch_types=[
            pltpu.VMEM((K,), jnp.int32),
            pltpu.VMEM((K,), jnp.int32),
            pltpu.VMEM((K,), jnp.int32),
            pltpu.VMEM((K,), jnp.int32),
            pltpu.VMEM((K, F), jnp.float32),
            pltpu.VMEM((K, F), jnp.float32),
            pltpu.VMEM_SHARED((NPAD, F), jnp.float32),
            pltpu.SemaphoreType.DMA,
            pltpu.SemaphoreType.DMA,
        ],
    )
    def k(table_h, ei_h, out_h, idx_s0, idx_s1, idx_d0, idx_d1,
          rows0, rows1, acc, gsem, ssem):
        cid = lax.axis_index("c")
        sid = lax.axis_index("s")
        wid = sid * NC + cid
        idx_s = [idx_s0, idx_s1]
        idx_d = [idx_d0, idx_d1]
        rows = [rows0, rows1]

        def zfill(i, _):
            rows0[i, pl.ds(0, 16)] = jnp.zeros((16,), jnp.float32)
            rows0[i, pl.ds(16, 16)] = jnp.zeros((16,), jnp.float32)
            return 0

        lax.fori_loop(0, ZR, zfill, 0)
        pltpu.sync_copy(rows0.at[pl.ds(0, ZR)], acc.at[pl.ds(sid * ZR, ZR)])
        plsc.subcore_barrier()

        base0 = wid * EPW

        def boff(i):
            return pl.multiple_of(base0 + i * K, 8)

        g = [None] * CH
        s = [None] * CH
        pltpu.sync_copy(ei_h.at[0, pl.ds(boff(0), K)], idx_s[0])
        g[0] = pltpu.async_copy(table_h.at[idx_s[0]], rows[0], gsem)
        for i in range(CH):
            j = i & 1
            pltpu.sync_copy(ei_h.at[1, pl.ds(boff(i), K)], idx_d[j])
            if i + 1 < CH:
                pltpu.sync_copy(ei_h.at[0, pl.ds(boff(i + 1), K)],
                                idx_s[1 - j])
            g[i].wait()
            s[i] = pltpu.async_copy(rows[j], acc.at[idx_d[j]], ssem,
                                    add=True)
            if i + 1 < CH:
                if i >= 1:
                    s[i - 1].wait()
                g[i + 1] = pltpu.async_copy(table_h.at[idx_s[1 - j]],
                                            rows[1 - j], gsem)
        if CH >= 2:
            s[CH - 2].wait()
        s[CH - 1].wait()
        plsc.subcore_barrier()
        pltpu.sync_copy(acc.at[pl.ds(sid * ZR, ZR)],
                        out_h.at[cid, pl.ds(sid * ZR, ZR)])

    return k(table, ei)


def _prep(x, W1, dop, ddp):
    N = x.shape[0]

    def body(x_ref, w_ref, do_ref, di_ref, t_ref, no_ref, ni_ref):
        do = do_ref[0, :N, :] + do_ref[1, :N, :]
        di = di_ref[0, :N, :] + di_ref[1, :N, :]
        no = jnp.where(do > 0, lax.rsqrt(jnp.maximum(do, 1e-12)), 0.0)
        ni = jnp.where(di > 0, lax.rsqrt(jnp.maximum(di, 1e-12)), 0.0)
        no_ref[...] = no
        ni_ref[...] = ni
        y = jnp.dot(x_ref[...], w_ref[...], preferred_element_type=jnp.float32)
        t_ref[...] = y * no[:, :1]

    return pl.pallas_call(
        body,
        out_shape=[
            jax.ShapeDtypeStruct((N, F), jnp.float32),
            jax.ShapeDtypeStruct((N, HW), jnp.float32),
            jax.ShapeDtypeStruct((N, HW), jnp.float32),
        ],
    )(x, W1, dop, ddp)


def _mid(parts, ni, no, b, Wn):
    N = ni.shape[0]

    def body(p_ref, ni_ref, no_ref, b_ref, w_ref, t_ref):
        agg = p_ref[0, :N, :] + p_ref[1, :N, :]
        h = jnp.maximum(agg * ni_ref[...][:, :1] + b_ref[...], 0.0)
        y = jnp.dot(h, w_ref[...], preferred_element_type=jnp.float32)
        t_ref[...] = y * no_ref[...][:, :1]

    return pl.pallas_call(
        body,
        out_shape=jax.ShapeDtypeStruct((N, F), jnp.float32),
    )(parts, ni, no, b, Wn)


def _final(parts, ni, b, Wfc, bfc):
    N = ni.shape[0]

    def body(p_ref, ni_ref, b_ref, w_ref, bf_ref, o_ref):
        agg = p_ref[0, :N, :] + p_ref[1, :N, :]
        h = jnp.maximum(agg * ni_ref[...][:, :1] + b_ref[...], 0.0)
        pooled = jnp.max(h, axis=0, keepdims=True)
        logits = jnp.dot(pooled, w_ref[...],
                         preferred_element_type=jnp.float32) + bf_ref[...]
        m = jnp.max(logits, axis=-1, keepdims=True)
        e = jnp.exp(logits - m)
        o_ref[...] = e / jnp.sum(e, axis=-1, keepdims=True)

    return pl.pallas_call(
        body,
        out_shape=jax.ShapeDtypeStruct((1, bfc.shape[-1]), jnp.float32),
    )(parts, ni, b, Wfc, bfc)


def kernel(inputs, edge_index, W1, b1, W2, b2, W3, b3, Wfc, bfc):
    dop, ddp = _hist(edge_index)
    table1, no, ni = _prep(inputs, W1, dop, ddp)
    p1 = _edge_agg(table1, edge_index)
    table2 = _mid(p1, ni, no, b1, W2)
    p2 = _edge_agg(table2, edge_index)
    table3 = _mid(p2, ni, no, b2, W3)
    p3 = _edge_agg(table3, edge_index)
    return _final(p3, ni, b3, Wfc, bfc)


# speedup vs baseline: 30.7474x; 1.4110x over previous
import functools

import jax
import jax.numpy as jnp
from jax import lax
from jax.experimental import pallas as pl
from jax.experimental.pallas import tpu as pltpu
from jax.experimental.pallas import tpu_sc as plsc

NC, NS = 2, 16
NW = NC * NS
NPAD = 10240
ZR = NPAD // NS
HW = 16
F = 32
K = 1000


def _sc_mesh():
    return plsc.VectorSubcoreMesh(
        core_axis_name="c", subcore_axis_name="s", num_cores=NC, num_subcores=NS
    )


_SC_PARAMS = pltpu.CompilerParams(use_tc_tiling_on_sc=False)


def _hist(ei3):
    E = ei3.shape[1] * ei3.shape[2]
    EPW = E // NW
    CH = EPW // K

    @functools.partial(
        pl.kernel,
        out_type=[
            jax.ShapeDtypeStruct((NC, NPAD, F), jnp.float32),
            jax.ShapeDtypeStruct((NC, NPAD, F), jnp.float32),
        ],
        mesh=_sc_mesh(),
        compiler_params=_SC_PARAMS,
        scratch_types=[
            pltpu.VMEM((CH, K), jnp.int32),
            pltpu.VMEM((CH, K), jnp.int32),
            pltpu.VMEM((K, HW), jnp.float32),
            pltpu.VMEM((ZR, HW), jnp.float32),
            pltpu.VMEM((ZR, F), jnp.float32),
            pltpu.VMEM_SHARED((NPAD, HW), jnp.float32),
            pltpu.VMEM_SHARED((NPAD, HW), jnp.float32),
            pltpu.SemaphoreType.DMA,
            pltpu.SemaphoreType.DMA,
        ],
    )
    def k(ei_h, out_o, out_i, idx_a, idx_b, ones, zbuf, wide,
          acc_o, acc_i, sem_o, sem_i):
        cid = lax.axis_index("c")
        sid = lax.axis_index("s")
        wid = sid * NC + cid

        def fill(i, _):
            ones[i, :] = jnp.ones((HW,), jnp.float32)
            zbuf[i % ZR, :] = jnp.zeros((HW,), jnp.float32)
            return 0

        lax.fori_loop(0, K, fill, 0)
        pltpu.sync_copy(zbuf, acc_o.at[pl.ds(sid * ZR, ZR)])
        pltpu.sync_copy(zbuf, acc_i.at[pl.ds(sid * ZR, ZR)])
        rb = wid * CH
        pltpu.sync_copy(ei_h.at[0, pl.ds(rb, CH)], idx_a)
        pltpu.sync_copy(ei_h.at[1, pl.ds(rb, CH)], idx_b)
        plsc.subcore_barrier()

        so = [None] * CH
        si = [None] * CH
        for i in range(CH):
            so[i] = pltpu.async_copy(ones, acc_o.at[idx_a.at[i]], sem_o,
                                     add=True)
            si[i] = pltpu.async_copy(ones, acc_i.at[idx_b.at[i]], sem_i,
                                     add=True)
        for i in range(CH):
            so[i].wait()
            si[i].wait()
        plsc.subcore_barrier()

        for acc, out in ((acc_o, out_o), (acc_i, out_i)):
            pltpu.sync_copy(acc.at[pl.ds(sid * ZR, ZR)], zbuf)

            def dup(r, _):
                v = zbuf[r, :]
                wide[r, pl.ds(0, HW)] = v
                wide[r, pl.ds(HW, HW)] = v
                return 0

            lax.fori_loop(0, ZR, dup, 0)
            pltpu.sync_copy(wide, out.at[cid, pl.ds(sid * ZR, ZR)])

    return k(ei3)


def _edge_agg(table, ei3):
    E = ei3.shape[1] * ei3.shape[2]
    EPW = E // NW
    CH = EPW // K

    @functools.partial(
        pl.kernel,
        out_type=jax.ShapeDtypeStruct((NC, NPAD, F), jnp.float32),
        mesh=_sc_mesh(),
        compiler_params=_SC_PARAMS,
        scratch_types=[
            pltpu.VMEM((CH, K), jnp.int32),
            pltpu.VMEM((CH, K), jnp.int32),
            pltpu.VMEM((K, F), jnp.float32),
            pltpu.VMEM((K, F), jnp.float32),
            pltpu.VMEM_SHARED((NPAD, F), jnp.float32),
            pltpu.SemaphoreType.DMA,
            pltpu.SemaphoreType.DMA,
        ],
    )
    def k(table_h, ei_h, out_h, idx_s, idx_d, rows0, rows1,
          acc, gsem, ssem):
        cid = lax.axis_index("c")
        sid = lax.axis_index("s")
        wid = sid * NC + cid
        rows = [rows0, rows1]

        def zfill(i, _):
            rows0[i, pl.ds(0, 16)] = jnp.zeros((16,), jnp.float32)
            rows0[i, pl.ds(16, 16)] = jnp.zeros((16,), jnp.float32)
            return 0

        lax.fori_loop(0, ZR, zfill, 0)
        pltpu.sync_copy(rows0.at[pl.ds(0, ZR)], acc.at[pl.ds(sid * ZR, ZR)])
        rb = wid * CH
        pltpu.sync_copy(ei_h.at[0, pl.ds(rb, CH)], idx_s)
        pltpu.sync_copy(ei_h.at[1, pl.ds(rb, CH)], idx_d)
        plsc.subcore_barrier()

        g = [None] * CH
        s = [None] * CH
        g[0] = pltpu.async_copy(table_h.at[idx_s.at[0]], rows[0], gsem)
        for i in range(CH):
            j = i & 1
            g[i].wait()
            s[i] = pltpu.async_copy(rows[j], acc.at[idx_d.at[i]], ssem,
                                    add=True)
            if i + 1 < CH:
                if i >= 1:
                    s[i - 1].wait()
                g[i + 1] = pltpu.async_copy(table_h.at[idx_s.at[i + 1]],
                                            rows[1 - j], gsem)
        if CH >= 2:
            s[CH - 2].wait()
        s[CH - 1].wait()
        plsc.subcore_barrier()
        pltpu.sync_copy(acc.at[pl.ds(sid * ZR, ZR)],
                        out_h.at[cid, pl.ds(sid * ZR, ZR)])

    return k(table, ei3)


def _prep(xp, w1bd, dop, ddp):
    RP = xp.shape[0]
    RN = dop.shape[1]

    def body(x_ref, w_ref, do_ref, di_ref, t_ref, no_ref, ni_ref):
        do = do_ref[0] + do_ref[1]
        di = di_ref[0] + di_ref[1]
        no = jnp.where(do > 0, lax.rsqrt(jnp.maximum(do, 1e-12)), 0.0)
        ni = jnp.where(di > 0, lax.rsqrt(jnp.maximum(di, 1e-12)), 0.0)
        no_ref[...] = no
        ni_ref[...] = ni
        y = jnp.dot(x_ref[...], w_ref[...], preferred_element_type=jnp.float32)
        t_ref[...] = y * no[:RP]

    return pl.pallas_call(
        body,
        out_shape=[
            jax.ShapeDtypeStruct((RP, 128), jnp.float32),
            jax.ShapeDtypeStruct((RN, 128), jnp.float32),
            jax.ShapeDtypeStruct((RN, 128), jnp.float32),
        ],
    )(xp, w1bd, dop, ddp)


def _mid(parts, nip, nop, bp, wbd):
    RP = NPAD // 4 - (NPAD - 10000) // 4

    def body(p_ref, ni_ref, no_ref, b_ref, w_ref, t_ref):
        agg = p_ref[0, :RP] + p_ref[1, :RP]
        h = jnp.maximum(agg * ni_ref[:RP] + b_ref[...], 0.0)
        y = jnp.dot(h, w_ref[...], preferred_element_type=jnp.float32)
        t_ref[...] = y * no_ref[:RP]

    return pl.pallas_call(
        body,
        out_shape=jax.ShapeDtypeStruct((RP, 128), jnp.float32),
    )(parts, nip, nop, bp, wbd)


def _final(parts, nip, bp, Wfc, bfc):
    RP = 2500

    def body(p_ref, ni_ref, b_ref, w_ref, bf_ref, o_ref):
        agg = p_ref[0, :RP] + p_ref[1, :RP]
        h = jnp.maximum(agg * ni_ref[:RP] + b_ref[...], 0.0)
        m = jnp.max(h, axis=0, keepdims=True)
        m32 = jnp.maximum(jnp.maximum(m[:, 0:32], m[:, 32:64]),
                          jnp.maximum(m[:, 64:96], m[:, 96:128]))
        logits = jnp.dot(m32, w_ref[...],
                         preferred_element_type=jnp.float32) + bf_ref[...]
        mx = jnp.max(logits, axis=-1, keepdims=True)
        e = jnp.exp(logits - mx)
        o_ref[...] = e / jnp.sum(e, axis=-1, keepdims=True)

    return pl.pallas_call(
        body,
        out_shape=jax.ShapeDtypeStruct((1, bfc.shape[-1]), jnp.float32),
    )(parts, nip, bp, Wfc, bfc)


def kernel(inputs, edge_index, W1, b1, W2, b2, W3, b3, Wfc, bfc):
    N = inputs.shape[0]
    E = edge_index.shape[1]
    ei3 = edge_index.reshape(2, E // K, K)
    xp = inputs.reshape(N // 4, 512)
    eye4 = jnp.eye(4, dtype=jnp.float32)
    w1bd = jnp.kron(eye4, W1)
    w2bd = jnp.kron(eye4, W2)
    w3bd = jnp.kron(eye4, W3)
    b1p = jnp.tile(b1, 4).reshape(1, 128)
    b2p = jnp.tile(b2, 4).reshape(1, 128)
    b3p = jnp.tile(b3, 4).reshape(1, 128)
    bfc2 = bfc.reshape(1, -1)

    dop, ddp = _hist(ei3)
    dopP = dop.reshape(NC, NPAD // 4, 128)
    ddpP = ddp.reshape(NC, NPAD // 4, 128)
    t1p, nop, nip = _prep(xp, w1bd, dopP, ddpP)

    p1 = _edge_agg(t1p.reshape(N, F), ei3).reshape(NC, NPAD // 4, 128)
    t2p = _mid(p1, nip, nop, b1p, w2bd)
    p2 = _edge_agg(t2p.reshape(N, F), ei3).reshape(NC, NPAD // 4, 128)
    t3p = _mid(p2, nip, nop, b2p, w3bd)
    p3 = _edge_agg(t3p.reshape(N, F), ei3).reshape(NC, NPAD // 4, 128)
    return _final(p3, nip, b3p, Wfc, bfc2)


# speedup vs baseline: 32.1664x; 1.0461x over previous
import functools

import jax
import jax.numpy as jnp
from jax import lax
from jax.experimental import pallas as pl
from jax.experimental.pallas import tpu as pltpu
from jax.experimental.pallas import tpu_sc as plsc

NC, NS = 2, 16
NW = NC * NS
NPAD = 10240
ZR = NPAD // NS
HW = 16
F = 32
K = 500


def _sc_mesh():
    return plsc.VectorSubcoreMesh(
        core_axis_name="c", subcore_axis_name="s", num_cores=NC, num_subcores=NS
    )


_SC_PARAMS = pltpu.CompilerParams(use_tc_tiling_on_sc=False)


def _hist(ei3):
    E = ei3.shape[1] * ei3.shape[2]
    EPW = E // NW
    CH = EPW // K

    @functools.partial(
        pl.kernel,
        out_type=[
            jax.ShapeDtypeStruct((NC, NPAD, F), jnp.float32),
            jax.ShapeDtypeStruct((NC, NPAD, F), jnp.float32),
        ],
        mesh=_sc_mesh(),
        compiler_params=_SC_PARAMS,
        scratch_types=[
            pltpu.VMEM((CH, K), jnp.int32),
            pltpu.VMEM((CH, K), jnp.int32),
            pltpu.VMEM((K, HW), jnp.float32),
            pltpu.VMEM((ZR, HW), jnp.float32),
            pltpu.VMEM((ZR, F), jnp.float32),
            pltpu.VMEM_SHARED((NPAD, HW), jnp.float32),
            pltpu.VMEM_SHARED((NPAD, HW), jnp.float32),
            pltpu.SemaphoreType.DMA,
            pltpu.SemaphoreType.DMA,
        ],
    )
    def k(ei_h, out_o, out_i, idx_a, idx_b, ones, zbuf, wide,
          acc_o, acc_i, sem_o, sem_i):
        cid = lax.axis_index("c")
        sid = lax.axis_index("s")
        wid = sid * NC + cid

        def fill(i, _):
            ones[i, :] = jnp.ones((HW,), jnp.float32)
            zbuf[i % ZR, :] = jnp.zeros((HW,), jnp.float32)
            return 0

        lax.fori_loop(0, K, fill, 0)
        pltpu.sync_copy(zbuf, acc_o.at[pl.ds(sid * ZR, ZR)])
        pltpu.sync_copy(zbuf, acc_i.at[pl.ds(sid * ZR, ZR)])
        rb = wid * CH
        pltpu.sync_copy(ei_h.at[0, pl.ds(rb, CH)], idx_a)
        pltpu.sync_copy(ei_h.at[1, pl.ds(rb, CH)], idx_b)
        plsc.subcore_barrier()

        so = [None] * CH
        si = [None] * CH
        for i in range(CH):
            so[i] = pltpu.async_copy(ones, acc_o.at[idx_a.at[i]], sem_o,
                                     add=True)
            si[i] = pltpu.async_copy(ones, acc_i.at[idx_b.at[i]], sem_i,
                                     add=True)
        for i in range(CH):
            so[i].wait()
            si[i].wait()
        plsc.subcore_barrier()

        for acc, out in ((acc_o, out_o), (acc_i, out_i)):
            pltpu.sync_copy(acc.at[pl.ds(sid * ZR, ZR)], zbuf)

            def dup(r, _):
                v = zbuf[r, :]
                wide[r, pl.ds(0, HW)] = v
                wide[r, pl.ds(HW, HW)] = v
                return 0

            lax.fori_loop(0, ZR, dup, 0)
            pltpu.sync_copy(wide, out.at[cid, pl.ds(sid * ZR, ZR)])

    return k(ei3)


def _edge_agg(table, ei3):
    E = ei3.shape[1] * ei3.shape[2]
    EPW = E // NW
    CH = EPW // K

    @functools.partial(
        pl.kernel,
        out_type=jax.ShapeDtypeStruct((NC, NPAD, F), jnp.float32),
        mesh=_sc_mesh(),
        compiler_params=_SC_PARAMS,
        scratch_types=[
            pltpu.VMEM((CH, K), jnp.int32),
            pltpu.VMEM((CH, K), jnp.int32),
            pltpu.VMEM((K, F), jnp.float32),
            pltpu.VMEM((K, F), jnp.float32),
            pltpu.VMEM((K, F), jnp.float32),
            pltpu.VMEM_SHARED((NPAD, F), jnp.float32),
            pltpu.SemaphoreType.DMA,
            pltpu.SemaphoreType.DMA,
        ],
    )
    def k(table_h, ei_h, out_h, idx_s, idx_d, rows0, rows1, rows2,
          acc, gsem, ssem):
        cid = lax.axis_index("c")
        sid = lax.axis_index("s")
        wid = sid * NC + cid
        rows = [rows0, rows1, rows2]

        def zfill(i, _):
            rows0[i, pl.ds(0, 16)] = jnp.zeros((16,), jnp.float32)
            rows0[i, pl.ds(16, 16)] = jnp.zeros((16,), jnp.float32)
            return 0

        lax.fori_loop(0, ZR, zfill, 0)
        pltpu.sync_copy(rows0.at[pl.ds(0, ZR)], acc.at[pl.ds(sid * ZR, ZR)])
        rb = wid * CH
        pltpu.sync_copy(ei_h.at[0, pl.ds(rb, CH)], idx_s)
        pltpu.sync_copy(ei_h.at[1, pl.ds(rb, CH)], idx_d)
        plsc.subcore_barrier()

        g = [None] * CH
        s = [None] * CH
        g[0] = pltpu.async_copy(table_h.at[idx_s.at[0]], rows[0], gsem)
        if CH >= 2:
            g[1] = pltpu.async_copy(table_h.at[idx_s.at[1]], rows[1], gsem)
        for i in range(CH):
            g[i].wait()
            s[i] = pltpu.async_copy(rows[i % 3], acc.at[idx_d.at[i]], ssem,
                                    add=True)
            if i + 2 < CH:
                if i >= 1:
                    s[i - 1].wait()
                g[i + 2] = pltpu.async_copy(table_h.at[idx_s.at[i + 2]],
                                            rows[(i + 2) % 3], gsem)
        if CH >= 3:
            s[CH - 3].wait()
        if CH >= 2:
            s[CH - 2].wait()
        s[CH - 1].wait()
        plsc.subcore_barrier()
        pltpu.sync_copy(acc.at[pl.ds(sid * ZR, ZR)],
                        out_h.at[cid, pl.ds(sid * ZR, ZR)])

    return k(table, ei3)


def _prep(xp, w1bd, dop, ddp):
    RP = xp.shape[0]
    RN = dop.shape[1]

    def body(x_ref, w_ref, do_ref, di_ref, t_ref, no_ref, ni_ref):
        do = do_ref[0] + do_ref[1]
        di = di_ref[0] + di_ref[1]
        no = jnp.where(do > 0, lax.rsqrt(jnp.maximum(do, 1e-12)), 0.0)
        ni = jnp.where(di > 0, lax.rsqrt(jnp.maximum(di, 1e-12)), 0.0)
        no_ref[...] = no
        ni_ref[...] = ni
        y = jnp.dot(x_ref[...], w_ref[...], preferred_element_type=jnp.float32)
        t_ref[...] = y * no[:RP]

    return pl.pallas_call(
        body,
        out_shape=[
            jax.ShapeDtypeStruct((RP, 128), jnp.float32),
            jax.ShapeDtypeStruct((RN, 128), jnp.float32),
            jax.ShapeDtypeStruct((RN, 128), jnp.float32),
        ],
    )(xp, w1bd, dop, ddp)


def _mid(parts, nip, nop, bp, wbd):
    RP = NPAD // 4 - (NPAD - 10000) // 4

    def body(p_ref, ni_ref, no_ref, b_ref, w_ref, t_ref):
        agg = p_ref[0, :RP] + p_ref[1, :RP]
        b = b_ref[...]
        bp = jnp.concatenate([b, b, b, b])
        h = jnp.maximum(agg * ni_ref[:RP] + bp, 0.0)
        y = jnp.dot(h, w_ref[...], preferred_element_type=jnp.float32)
        t_ref[...] = y * no_ref[:RP]

    return pl.pallas_call(
        body,
        out_shape=jax.ShapeDtypeStruct((RP, 128), jnp.float32),
    )(parts, nip, nop, bp, wbd)


def _final(parts, nip, bp, Wfc, bfc):
    RP = 2500

    def body(p_ref, ni_ref, b_ref, w_ref, bf_ref, o_ref):
        agg = p_ref[0, :RP] + p_ref[1, :RP]
        b = b_ref[...]
        bp = jnp.concatenate([b, b, b, b])
        h = jnp.maximum(agg * ni_ref[:RP] + bp, 0.0)
        m = jnp.max(h, axis=0, keepdims=True)
        m32 = jnp.maximum(jnp.maximum(m[:, 0:32], m[:, 32:64]),
                          jnp.maximum(m[:, 64:96], m[:, 96:128]))
        logits = jnp.dot(m32, w_ref[...],
                         preferred_element_type=jnp.float32) + bf_ref[...]
        mx = jnp.max(logits, axis=-1, keepdims=True)
        e = jnp.exp(logits - mx)
        o_ref[...] = e / jnp.sum(e, axis=-1, keepdims=True)

    return pl.pallas_call(
        body,
        out_shape=jax.ShapeDtypeStruct((1, bfc.shape[-1]), jnp.float32),
    )(parts, nip, bp, Wfc, bfc)


def kernel(inputs, edge_index, W1, b1, W2, b2, W3, b3, Wfc, bfc):
    N = inputs.shape[0]
    E = edge_index.shape[1]
    ei3 = edge_index.reshape(2, E // K, K)
    xp = inputs.reshape(N // 4, 512)
    eye4 = jnp.eye(4, dtype=jnp.float32)
    w1bd = jnp.kron(eye4, W1)
    w2bd = jnp.kron(eye4, W2)
    w3bd = jnp.kron(eye4, W3)
    bfc2 = bfc.reshape(1, -1)

    dop, ddp = _hist(ei3)
    dopP = dop.reshape(NC, NPAD // 4, 128)
    ddpP = ddp.reshape(NC, NPAD // 4, 128)
    t1p, nop, nip = _prep(xp, w1bd, dopP, ddpP)

    p1 = _edge_agg(t1p.reshape(N, F), ei3).reshape(NC, NPAD // 4, 128)
    t2p = _mid(p1, nip, nop, b1, w2bd)
    p2 = _edge_agg(t2p.reshape(N, F), ei3).reshape(NC, NPAD // 4, 128)
    t3p = _mid(p2, nip, nop, b2, w3bd)
    p3 = _edge_agg(t3p.reshape(N, F), ei3).reshape(NC, NPAD // 4, 128)
    return _final(p3, nip, b3, Wfc, bfc2)


# speedup vs baseline: 32.3192x; 1.0048x over previous
import functools

import jax
import jax.numpy as jnp
from jax import lax
from jax.experimental import pallas as pl
from jax.experimental.pallas import tpu as pltpu
from jax.experimental.pallas import tpu_sc as plsc

NC, NS = 2, 16
NW = NC * NS
NPAD = 10240
ZR = NPAD // NS
HW = 16
F = 32
K = 500


def _sc_mesh():
    return plsc.VectorSubcoreMesh(
        core_axis_name="c", subcore_axis_name="s", num_cores=NC, num_subcores=NS
    )


_SC_PARAMS = pltpu.CompilerParams(use_tc_tiling_on_sc=False)


def _hist(ei3):
    E = ei3.shape[1] * ei3.shape[2]
    EPW = E // NW
    CH = EPW // K

    @functools.partial(
        pl.kernel,
        out_type=[
            jax.ShapeDtypeStruct((NC, NPAD, F), jnp.float32),
            jax.ShapeDtypeStruct((NC, NPAD, F), jnp.float32),
        ],
        mesh=_sc_mesh(),
        compiler_params=_SC_PARAMS,
        scratch_types=[
            pltpu.VMEM((CH, K), jnp.int32),
            pltpu.VMEM((CH, K), jnp.int32),
            pltpu.VMEM((K, HW), jnp.float32),
            pltpu.VMEM((ZR, HW), jnp.float32),
            pltpu.VMEM((ZR, F), jnp.float32),
            pltpu.VMEM_SHARED((NPAD, HW), jnp.float32),
            pltpu.VMEM_SHARED((NPAD, HW), jnp.float32),
            pltpu.SemaphoreType.DMA,
            pltpu.SemaphoreType.DMA,
        ],
    )
    def k(ei_h, out_o, out_i, idx_a, idx_b, ones, zbuf, wide,
          acc_o, acc_i, sem_o, sem_i):
        cid = lax.axis_index("c")
        sid = lax.axis_index("s")
        wid = sid * NC + cid

        def fill(i, _):
            ones[i, :] = jnp.ones((HW,), jnp.float32)
            zbuf[i % ZR, :] = jnp.zeros((HW,), jnp.float32)
            return 0

        lax.fori_loop(0, K, fill, 0)
        pltpu.sync_copy(zbuf, acc_o.at[pl.ds(sid * ZR, ZR)])
        pltpu.sync_copy(zbuf, acc_i.at[pl.ds(sid * ZR, ZR)])
        rb = wid * CH
        pltpu.sync_copy(ei_h.at[0, pl.ds(rb, CH)], idx_a)
        pltpu.sync_copy(ei_h.at[1, pl.ds(rb, CH)], idx_b)
        plsc.subcore_barrier()

        so = [None] * CH
        si = [None] * CH
        for i in range(CH):
            so[i] = pltpu.async_copy(ones, acc_o.at[idx_a.at[i]], sem_o,
                                     add=True)
            si[i] = pltpu.async_copy(ones, acc_i.at[idx_b.at[i]], sem_i,
                                     add=True)
        for i in range(CH):
            so[i].wait()
            si[i].wait()
        plsc.subcore_barrier()

        for acc, out in ((acc_o, out_o), (acc_i, out_i)):
            pltpu.sync_copy(acc.at[pl.ds(sid * ZR, ZR)], zbuf)

            def dup(r, _):
                v = zbuf[r, :]
                wide[r, pl.ds(0, HW)] = v
                wide[r, pl.ds(HW, HW)] = v
                return 0

            lax.fori_loop(0, ZR, dup, 0)
            pltpu.sync_copy(wide, out.at[cid, pl.ds(sid * ZR, ZR)])

    return k(ei3)


def _edge_agg(table, ei3):
    E = ei3.shape[1] * ei3.shape[2]
    EPW = E // NW
    CH = EPW // K

    @functools.partial(
        pl.kernel,
        out_type=jax.ShapeDtypeStruct((NC, NPAD, F), jnp.float32),
        mesh=_sc_mesh(),
        compiler_params=_SC_PARAMS,
        scratch_types=[
            pltpu.VMEM((CH, K), jnp.int32),
            pltpu.VMEM((CH, K), jnp.int32),
            pltpu.VMEM((K, F), jnp.float32),
            pltpu.VMEM((K, F), jnp.float32),
            pltpu.VMEM((K, F), jnp.float32),
            pltpu.VMEM_SHARED((NPAD, F), jnp.float32),
            pltpu.SemaphoreType.DMA,
            pltpu.SemaphoreType.DMA,
        ],
    )
    def k(table_h, ei_h, out_h, idx_s, idx_d, rows0, rows1, rows2,
          acc, gsem, ssem):
        cid = lax.axis_index("c")
        sid = lax.axis_index("s")
        wid = sid * NC + cid
        rows = [rows0, rows1, rows2]

        def zfill(i, _):
            rows0[i, pl.ds(0, 16)] = jnp.zeros((16,), jnp.float32)
            rows0[i, pl.ds(16, 16)] = jnp.zeros((16,), jnp.float32)
            return 0

        lax.fori_loop(0, ZR, zfill, 0)
        pltpu.sync_copy(rows0.at[pl.ds(0, ZR)], acc.at[pl.ds(sid * ZR, ZR)])
        rb = wid * CH
        pltpu.sync_copy(ei_h.at[0, pl.ds(rb, CH)], idx_s)
        pltpu.sync_copy(ei_h.at[1, pl.ds(rb, CH)], idx_d)
        plsc.subcore_barrier()

        g = [None] * CH
        s = [None] * CH
        g[0] = pltpu.async_copy(table_h.at[idx_s.at[0]], rows[0], gsem)
        if CH >= 2:
            g[1] = pltpu.async_copy(table_h.at[idx_s.at[1]], rows[1], gsem)
        for i in range(CH):
            g[i].wait()
            s[i] = pltpu.async_copy(rows[i % 3], acc.at[idx_d.at[i]], ssem,
                                    add=True)
            if i + 2 < CH:
                if i >= 1:
                    s[i - 1].wait()
                g[i + 2] = pltpu.async_copy(table_h.at[idx_s.at[i + 2]],
                                            rows[(i + 2) % 3], gsem)
        if CH >= 3:
            s[CH - 3].wait()
        if CH >= 2:
            s[CH - 2].wait()
        s[CH - 1].wait()
        plsc.subcore_barrier()
        pltpu.sync_copy(acc.at[pl.ds(sid * ZR, ZR)],
                        out_h.at[cid, pl.ds(sid * ZR, ZR)])

    return k(table, ei3)


def _proj(xp, w1bd):
    RP = xp.shape[0]

    def body(x_ref, w_ref, y_ref):
        y_ref[...] = jnp.dot(x_ref[...], w_ref[...],
                             preferred_element_type=jnp.float32)

    return pl.pallas_call(
        body,
        out_shape=jax.ShapeDtypeStruct((RP, 128), jnp.float32),
    )(xp, w1bd)


def _prep(y, dop, ddp):
    RP = y.shape[0]
    RN = dop.shape[1]

    def body(y_ref, do_ref, di_ref, t_ref, no_ref, ni_ref):
        do = do_ref[0] + do_ref[1]
        di = di_ref[0] + di_ref[1]
        no = jnp.where(do > 0, lax.rsqrt(jnp.maximum(do, 1e-12)), 0.0)
        ni = jnp.where(di > 0, lax.rsqrt(jnp.maximum(di, 1e-12)), 0.0)
        no_ref[...] = no
        ni_ref[...] = ni
        t_ref[...] = y_ref[...] * no[:RP]

    return pl.pallas_call(
        body,
        out_shape=[
            jax.ShapeDtypeStruct((RP, 128), jnp.float32),
            jax.ShapeDtypeStruct((RN, 128), jnp.float32),
            jax.ShapeDtypeStruct((RN, 128), jnp.float32),
        ],
    )(y, dop, ddp)


def _mid(parts, nip, nop, bp, wbd):
    RP = NPAD // 4 - (NPAD - 10000) // 4

    def body(p_ref, ni_ref, no_ref, b_ref, w_ref, t_ref):
        agg = p_ref[0, :RP] + p_ref[1, :RP]
        b = b_ref[...]
        bp = jnp.concatenate([b, b, b, b])
        h = jnp.maximum(agg * ni_ref[:RP] + bp, 0.0)
        y = jnp.dot(h, w_ref[...], preferred_element_type=jnp.float32)
        t_ref[...] = y * no_ref[:RP]

    return pl.pallas_call(
        body,
        out_shape=jax.ShapeDtypeStruct((RP, 128), jnp.float32),
    )(parts, nip, nop, bp, wbd)


def _final(parts, nip, bp, Wfc, bfc):
    RP = 2500

    def body(p_ref, ni_ref, b_ref, w_ref, bf_ref, o_ref):
        agg = p_ref[0, :RP] + p_ref[1, :RP]
        b = b_ref[...]
        bp = jnp.concatenate([b, b, b, b])
        h = jnp.maximum(agg * ni_ref[:RP] + bp, 0.0)
        m = jnp.max(h, axis=0, keepdims=True)
        m32 = jnp.maximum(jnp.maximum(m[:, 0:32], m[:, 32:64]),
                          jnp.maximum(m[:, 64:96], m[:, 96:128]))
        logits = jnp.dot(m32, w_ref[...],
                         preferred_element_type=jnp.float32) + bf_ref[...]
        mx = jnp.max(logits, axis=-1, keepdims=True)
        e = jnp.exp(logits - mx)
        o_ref[...] = e / jnp.sum(e, axis=-1, keepdims=True)

    return pl.pallas_call(
        body,
        out_shape=jax.ShapeDtypeStruct((1, bfc.shape[-1]), jnp.float32),
    )(parts, nip, bp, Wfc, bfc)


def kernel(inputs, edge_index, W1, b1, W2, b2, W3, b3, Wfc, bfc):
    N = inputs.shape[0]
    E = edge_index.shape[1]
    ei3 = edge_index.reshape(2, E // K, K)
    xp = inputs.reshape(N // 4, 512)
    eye4 = jnp.eye(4, dtype=jnp.float32)
    w1bd = jnp.kron(eye4, W1)
    w2bd = jnp.kron(eye4, W2)
    w3bd = jnp.kron(eye4, W3)
    bfc2 = bfc.reshape(1, -1)

    dop, ddp = _hist(ei3)
    y1 = _proj(xp, w1bd)
    dopP = dop.reshape(NC, NPAD // 4, 128)
    ddpP = ddp.reshape(NC, NPAD // 4, 128)
    t1p, nop, nip = _prep(y1, dopP, ddpP)

    p1 = _edge_agg(t1p.reshape(N, F), ei3).reshape(NC, NPAD // 4, 128)
    t2p = _mid(p1, nip, nop, b1, w2bd)
    p2 = _edge_agg(t2p.reshape(N, F), ei3).reshape(NC, NPAD // 4, 128)
    t3p = _mid(p2, nip, nop, b2, w3bd)
    p3 = _edge_agg(t3p.reshape(N, F), ei3).reshape(NC, NPAD // 4, 128)
    return _final(p3, nip, b3, Wfc, bfc2)
